# baseline (speedup 1.0000x reference)
import jax
import jax.numpy as jnp
from jax import lax
from jax.experimental import pallas as pl
from jax.experimental.pallas import tpu as pltpu
from jax.experimental.pallas import tpu_sc as plsc

N = 10000
E = 320000
D = 128
H = 8
DH = 16
OUT = 64
TS = 2 * D

NC = 2
NS = 16
NW = NC * NS
CH = 40
NCHUNK = E // CH
SPLIT = 624

_BLK = 1000
_GRID = N // _BLK

f32 = jnp.float32


def _emit_tables(wh, as_ref, ad_ref, ts_ref, td_ref, amax_ref):
    als = jnp.dot(wh, as_ref[...], preferred_element_type=f32)
    ald = jnp.dot(wh, ad_ref[...], preferred_element_type=f32)
    pad = jnp.zeros((_BLK, D - DH), f32)
    ts_ref[...] = jnp.concatenate([als, pad, wh], axis=1)
    td_ref[...] = jnp.concatenate([ald, pad], axis=1)
    blkmax = jnp.max(als, axis=0, keepdims=True)

    @pl.when(pl.program_id(0) == 0)
    def _():
        amax_ref[...] = blkmax

    @pl.when(pl.program_id(0) != 0)
    def _():
        amax_ref[...] = jnp.maximum(amax_ref[...], blkmax)


def _dense0_body(x_ref, win_ref, bin_ref, w1_ref, as_ref, ad_ref,
                 ts_ref, td_ref, amax_ref):
    h0 = jnp.dot(x_ref[...], win_ref[...], preferred_element_type=f32)
    h0 = h0 + bin_ref[...]
    wh = jnp.dot(h0, w1_ref[...], preferred_element_type=f32)
    _emit_tables(wh, as_ref, ad_ref, ts_ref, td_ref, amax_ref)


_TABLE_OUT_SPECS = [
    pl.BlockSpec((_BLK, TS), lambda i: (i, 0)),
    pl.BlockSpec((_BLK, D), lambda i: (i, 0)),
    pl.BlockSpec((1, DH), lambda i: (0, 0)),
]
_TABLE_OUT_SHAPE = [
    jax.ShapeDtypeStruct((N, TS), f32),
    jax.ShapeDtypeStruct((N, D), f32),
    jax.ShapeDtypeStruct((1, DH), f32),
]


def _dense0(x, W_in, b_in, W1, As, Ad):
    return pl.pallas_call(
        _dense0_body,
        grid=(_GRID,),
        in_specs=[
            pl.BlockSpec((_BLK, D), lambda i: (i, 0)),
            pl.BlockSpec((D, D), lambda i: (0, 0)),
            pl.BlockSpec((1, D), lambda i: (0, 0)),
            pl.BlockSpec((D, D), lambda i: (0, 0)),
            pl.BlockSpec((D, DH), lambda i: (0, 0)),
            pl.BlockSpec((D, DH), lambda i: (0, 0)),
        ],
        out_specs=_TABLE_OUT_SPECS,
        out_shape=_TABLE_OUT_SHAPE,
    )(x, W_in, b_in, W1, As, Ad)


def _dense1_body(agg_ref, den_ref, exp_ref, w2_ref, as_ref, ad_ref,
                 ts_ref, td_ref, amax_ref):
    agg = agg_ref[0] + agg_ref[1]
    den = den_ref[0] + den_ref[1]
    dexp = jnp.dot(den, exp_ref[...], preferred_element_type=f32)
    h = jnp.maximum(agg / (dexp + 1e-9), 0.0)
    wh = jnp.dot(h, w2_ref[...], preferred_element_type=f32)
    _emit_tables(wh, as_ref, ad_ref, ts_ref, td_ref, amax_ref)


def _dense1(agg, den, Exp, W2, As, Ad):
    return pl.pallas_call(
        _dense1_body,
        grid=(_GRID,),
        in_specs=[
            pl.BlockSpec((2, _BLK, D), lambda i: (0, i, 0)),
            pl.BlockSpec((2, _BLK, DH), lambda i: (0, i, 0)),
            pl.BlockSpec((DH, D), lambda i: (0, 0)),
            pl.BlockSpec((D, D), lambda i: (0, 0)),
            pl.BlockSpec((D, DH), lambda i: (0, 0)),
            pl.BlockSpec((D, DH), lambda i: (0, 0)),
        ],
        out_specs=_TABLE_OUT_SPECS,
        out_shape=_TABLE_OUT_SHAPE,
    )(agg, den, Exp, W2, As, Ad)


def _dense2_body(agg_ref, den_ref, exp_ref, wout_ref, bout_ref, o_ref):
    agg = agg_ref[0] + agg_ref[1]
    den = den_ref[0] + den_ref[1]
    dexp = jnp.dot(den, exp_ref[...], preferred_element_type=f32)
    h = jnp.maximum(agg / (dexp + 1e-9), 0.0)
    o_ref[...] = jnp.dot(h, wout_ref[...], preferred_element_type=f32) + bout_ref[...]


def _dense2(agg, den, Exp, W_out, b_out):
    return pl.pallas_call(
        _dense2_body,
        grid=(_GRID,),
        in_specs=[
            pl.BlockSpec((2, _BLK, D), lambda i: (0, i, 0)),
            pl.BlockSpec((2, _BLK, DH), lambda i: (0, i, 0)),
            pl.BlockSpec((DH, D), lambda i: (0, 0)),
            pl.BlockSpec((D, OUT), lambda i: (0, 0)),
            pl.BlockSpec((1, OUT), lambda i: (0, 0)),
        ],
        out_specs=pl.BlockSpec((_BLK, OUT), lambda i: (i, 0)),
        out_shape=jax.ShapeDtypeStruct((N, OUT), f32),
    )(agg, den, Exp, W_out, b_out)


def _edge_body(src_hbm, dst_hbm, ts_hbm, td_hbm, amax_hbm,
               agg_out, den_out,
               idx_s, idx_d, src_rows, ald_rows, msg_rows, w_rows, amax_vm,
               agg_acc, den_acc, sem_a, sem_b, sem_c):
    cid = lax.axis_index("c")
    sid = lax.axis_index("s")
    wid = sid * NC + cid

    zero16 = jnp.zeros((16,), f32)

    def _zero_msg(j, carry):
        for c in range(D // 16):
            msg_rows[j, pl.ds(c * 16, 16)] = zero16
        return carry

    lax.fori_loop(0, CH, _zero_msg, 0)

    def _zero_w(j, carry):
        w_rows[j] = zero16
        return carry

    lax.fori_loop(0, CH, _zero_w, 0)

    def _zero_acc(j, carry):
        base = pl.multiple_of(sid * SPLIT + j * 16, 16)
        pltpu.sync_copy(msg_rows.at[pl.ds(0, 16)], agg_acc.at[pl.ds(base, 16)])
        pltpu.sync_copy(w_rows.at[pl.ds(0, 16)], den_acc.at[pl.ds(base, 16)])
        return carry

    lax.fori_loop(0, SPLIT // 16, _zero_acc, 0)

    @pl.when(sid == NS - 1)
    def _():
        pltpu.sync_copy(msg_rows.at[pl.ds(0, 16)],
                        agg_acc.at[pl.ds(NS * SPLIT, 16)])
        pltpu.sync_copy(w_rows.at[pl.ds(0, 16)],
                        den_acc.at[pl.ds(NS * SPLIT, 16)])

    pltpu.async_copy(amax_hbm, amax_vm, sem_a).wait()
    amax_v = amax_vm[...]
    maskb = lax.broadcasted_iota(jnp.int32, (16,), 0) < H

    plsc.subcore_barrier()

    nch = jnp.where(wid < NCHUNK - (NCHUNK // NW) * NW, NCHUNK // NW + 1,
                    NCHUNK // NW)

    def _chunk(i, carry):
        off = (wid + NW * i) * CH
        pltpu.async_copy(src_hbm.at[pl.ds(off, CH)], idx_s, sem_a).wait()
        pltpu.async_copy(dst_hbm.at[pl.ds(off, CH)], idx_d, sem_b).wait()
        ca = pltpu.async_copy(ts_hbm.at[idx_s], src_rows, sem_a)
        cb = pltpu.async_copy(td_hbm.at[idx_d], ald_rows, sem_b)
        ca.wait()
        cb.wait()

        def _edge(k, carry2):
            als_v = src_rows[k, pl.ds(0, 16)]
            ald_v = ald_rows[k, pl.ds(0, 16)]
            t = als_v + ald_v
            e = jnp.maximum(t, 0.2 * t)
            u = amax_v + ald_v
            b = jnp.maximum(u, 0.2 * u)
            w = jnp.where(maskb, jnp.exp(e - b), 0.0)
            w_rows[k] = w
            for h in range(H):
                h_vec = jnp.full((16, 1), h, jnp.int32)
                wspl = lax.gather(
                    w, h_vec,
                    lax.GatherDimensionNumbers(
                        offset_dims=(), collapsed_slice_dims=(0,),
                        start_index_map=(0,)),
                    slice_sizes=(1,),
                    mode=lax.GatherScatterMode.PROMISE_IN_BOUNDS)
                part = src_rows[k, pl.ds(D + h * 16, 16)]
                msg_rows[k, pl.ds(h * 16, 16)] = part * wspl
            return carry2

        lax.fori_loop(0, CH, _edge, 0)

        pltpu.sync_copy(w_rows, den_acc.at[idx_d], add=True)
        pltpu.sync_copy(msg_rows, agg_acc.at[idx_d], add=True)
        return carry

    lax.fori_loop(0, nch, _chunk, 0)

    plsc.subcore_barrier()

    base = pl.multiple_of(sid * SPLIT, 16)
    pltpu.sync_copy(agg_acc.at[pl.ds(base, SPLIT)],
                    agg_out.at[cid, pl.ds(base, SPLIT)])
    pltpu.sync_copy(den_acc.at[pl.ds(base, SPLIT)],
                    den_out.at[cid, pl.ds(base, SPLIT)])

    @pl.when(sid == NS - 1)
    def _():
        pltpu.sync_copy(agg_acc.at[pl.ds(NS * SPLIT, 16)],
                        agg_out.at[cid, pl.ds(NS * SPLIT, 16)])
        pltpu.sync_copy(den_acc.at[pl.ds(NS * SPLIT, 16)],
                        den_out.at[cid, pl.ds(NS * SPLIT, 16)])


_edge_kernel = pl.kernel(
    _edge_body,
    out_type=(
        jax.ShapeDtypeStruct((NC, N, D), f32),
        jax.ShapeDtypeStruct((NC, N, DH), f32),
    ),
    mesh=plsc.VectorSubcoreMesh(core_axis_name="c", subcore_axis_name="s",
                                num_cores=NC, num_subcores=NS),
    scratch_types=(
        pltpu.VMEM((CH,), jnp.int32),
        pltpu.VMEM((CH,), jnp.int32),
        pltpu.VMEM((CH, TS), f32),
        pltpu.VMEM((CH, D), f32),
        pltpu.VMEM((CH, D), f32),
        pltpu.VMEM((CH, DH), f32),
        pltpu.VMEM((16,), f32),
        pltpu.VMEM_SHARED((N, D), f32),
        pltpu.VMEM_SHARED((N, DH), f32),
        pltpu.SemaphoreType.DMA,
        pltpu.SemaphoreType.DMA,
        pltpu.SemaphoreType.DMA,
    ),
)


def _attn_mat(a):
    m = (jnp.eye(H, dtype=f32)[:, None, :] * a[:, :, None]).reshape(D, H)
    return jnp.pad(m, ((0, 0), (0, DH - H)))


def _expand_mat():
    m = jnp.repeat(jnp.eye(H, dtype=f32), DH, axis=1)
    return jnp.pad(m, ((0, DH - H), (0, 0)))


def _edge_jnp(src, dst, ts, td, amax):
    als = ts[src, :DH]
    ald = td[dst, :DH]
    t = als + ald
    e = jnp.maximum(t, 0.2 * t)
    u = amax[None, :] + ald
    b = jnp.maximum(u, 0.2 * u)
    w = jnp.where(jnp.arange(DH) < H, jnp.exp(e - b), 0.0)
    wh = ts[src, D:]
    wexp = jnp.repeat(w[:, :H], DH, axis=1)
    agg = jax.ops.segment_sum(wh * wexp, dst, num_segments=N)
    den = jax.ops.segment_sum(w, dst, num_segments=N)
    aggp = jnp.stack([agg, jnp.zeros_like(agg)])
    denp = jnp.stack([den, jnp.zeros_like(den)])
    return aggp, denp


@jax.jit
def kernel(x, edge_index, W_in, b_in, W1, a_src1, a_dst1, W2, a_src2, a_dst2,
           W_out, b_out):
    src = edge_index[0]
    dst = edge_index[1]
    Exp = _expand_mat()

    ts1, td1, amax1 = _dense0(x, W_in, b_in.reshape(1, D), W1,
                              _attn_mat(a_src1), _attn_mat(a_dst1))
    agg1, den1 = _edge_jnp(src, dst, ts1, td1, amax1.reshape(DH))
    ts2, td2, amax2 = _dense1(agg1, den1, Exp, W2,
                              _attn_mat(a_src2), _attn_mat(a_dst2))
    agg2, den2 = _edge_jnp(src, dst, ts2, td2, amax2.reshape(DH))
    return _dense2(agg2, den2, Exp, W_out, b_out.reshape(1, OUT))


# speedup vs baseline: 100.3762x; 100.3762x over previous
import jax
import jax.numpy as jnp
from jax import lax
from jax.experimental import pallas as pl
from jax.experimental.pallas import tpu as pltpu
from jax.experimental.pallas import tpu_sc as plsc

N = 10000
E = 320000
D = 128
H = 8
DH = 16
OUT = 64
TS = 2 * D

NC = 2
NS = 16
NW = NC * NS
CH = 16
NCHUNK = E // CH
NPAD = 10240
NPG = NPAD // 8

_BLK = 1000
_GRID = N // _BLK

f32 = jnp.float32


def _emit_tables(wh, as_ref, ad_ref, ts_ref, td_ref, amax_ref):
    als = jnp.dot(wh, as_ref[...], preferred_element_type=f32)
    ald = jnp.dot(wh, ad_ref[...], preferred_element_type=f32)
    pad = jnp.zeros((_BLK, D - DH), f32)
    ts_ref[...] = jnp.concatenate([als, pad, wh], axis=1)
    td_ref[...] = jnp.concatenate([ald, pad], axis=1)
    blkmax = jnp.max(als, axis=0, keepdims=True)

    @pl.when(pl.program_id(0) == 0)
    def _():
        amax_ref[...] = blkmax

    @pl.when(pl.program_id(0) != 0)
    def _():
        amax_ref[...] = jnp.maximum(amax_ref[...], blkmax)


def _dense0_body(x_ref, win_ref, bin_ref, w1_ref, as_ref, ad_ref,
                 ts_ref, td_ref, amax_ref):
    h0 = jnp.dot(x_ref[...], win_ref[...], preferred_element_type=f32)
    h0 = h0 + bin_ref[...]
    wh = jnp.dot(h0, w1_ref[...], preferred_element_type=f32)
    _emit_tables(wh, as_ref, ad_ref, ts_ref, td_ref, amax_ref)


_TABLE_OUT_SPECS = [
    pl.BlockSpec((_BLK, TS), lambda i: (i, 0)),
    pl.BlockSpec((_BLK, D), lambda i: (i, 0)),
    pl.BlockSpec((1, DH), lambda i: (0, 0)),
]
_TABLE_OUT_SHAPE = [
    jax.ShapeDtypeStruct((N, TS), f32),
    jax.ShapeDtypeStruct((N, D), f32),
    jax.ShapeDtypeStruct((1, DH), f32),
]


def _dense0(x, W_in, b_in, W1, As, Ad):
    return pl.pallas_call(
        _dense0_body,
        grid=(_GRID,),
        in_specs=[
            pl.BlockSpec((_BLK, D), lambda i: (i, 0)),
            pl.BlockSpec((D, D), lambda i: (0, 0)),
            pl.BlockSpec((1, D), lambda i: (0, 0)),
            pl.BlockSpec((D, D), lambda i: (0, 0)),
            pl.BlockSpec((D, DH), lambda i: (0, 0)),
            pl.BlockSpec((D, DH), lambda i: (0, 0)),
        ],
        out_specs=_TABLE_OUT_SPECS,
        out_shape=_TABLE_OUT_SHAPE,
    )(x, W_in, b_in, W1, As, Ad)


def _dense1_body(agg_ref, den_ref, exp_ref, w2_ref, as_ref, ad_ref,
                 ts_ref, td_ref, amax_ref):
    agg = agg_ref[0] + agg_ref[1]
    den = den_ref[0] + den_ref[1]
    dexp = jnp.dot(den, exp_ref[...], preferred_element_type=f32)
    h = jnp.maximum(agg / (dexp + 1e-9), 0.0)
    wh = jnp.dot(h, w2_ref[...], preferred_element_type=f32)
    _emit_tables(wh, as_ref, ad_ref, ts_ref, td_ref, amax_ref)


def _dense1(agg, den, Exp, W2, As, Ad):
    return pl.pallas_call(
        _dense1_body,
        grid=(_GRID,),
        in_specs=[
            pl.BlockSpec((2, _BLK, D), lambda i: (0, i, 0)),
            pl.BlockSpec((2, _BLK, DH), lambda i: (0, i, 0)),
            pl.BlockSpec((DH, D), lambda i: (0, 0)),
            pl.BlockSpec((D, D), lambda i: (0, 0)),
            pl.BlockSpec((D, DH), lambda i: (0, 0)),
            pl.BlockSpec((D, DH), lambda i: (0, 0)),
        ],
        out_specs=_TABLE_OUT_SPECS,
        out_shape=_TABLE_OUT_SHAPE,
    )(agg, den, Exp, W2, As, Ad)


def _dense2_body(agg_ref, den_ref, exp_ref, wout_ref, bout_ref, o_ref):
    agg = agg_ref[0] + agg_ref[1]
    den = den_ref[0] + den_ref[1]
    dexp = jnp.dot(den, exp_ref[...], preferred_element_type=f32)
    h = jnp.maximum(agg / (dexp + 1e-9), 0.0)
    o_ref[...] = jnp.dot(h, wout_ref[...], preferred_element_type=f32) + bout_ref[...]


def _dense2(agg, den, Exp, W_out, b_out):
    return pl.pallas_call(
        _dense2_body,
        grid=(_GRID,),
        in_specs=[
            pl.BlockSpec((2, _BLK, D), lambda i: (0, i, 0)),
            pl.BlockSpec((2, _BLK, DH), lambda i: (0, i, 0)),
            pl.BlockSpec((DH, D), lambda i: (0, 0)),
            pl.BlockSpec((D, OUT), lambda i: (0, 0)),
            pl.BlockSpec((1, OUT), lambda i: (0, 0)),
        ],
        out_specs=pl.BlockSpec((_BLK, OUT), lambda i: (i, 0)),
        out_shape=jax.ShapeDtypeStruct((N, OUT), f32),
    )(agg, den, Exp, W_out, b_out)


def _edge_body(src_hbm, dst_hbm, ts_hbm, td_hbm, amax_hbm,
               agg_out, den_out,
               idx_s, idx_d, idx_dz, idx_z, src_rows, ald_rows,
               msg_rows, den_rows, amax_vm, agg_acc, den_acc, sem_a, sem_b):
    cid = lax.axis_index("c")
    sid = lax.axis_index("s")
    wid = sid * NC + cid

    zero16 = jnp.zeros((16,), f32)
    iota16 = lax.broadcasted_iota(jnp.int32, (16,), 0)

    def _zero_msg(j, carry):
        for c in range(D // 16):
            msg_rows[j, pl.ds(c * 16, 16)] = zero16
        return carry

    lax.fori_loop(0, CH, _zero_msg, 0)

    def _zero_den(j, carry):
        for c in range(D // 16):
            den_rows[j, pl.ds(c * 16, 16)] = zero16
        return carry

    lax.fori_loop(0, CH, _zero_den, 0)

    def _zero_acc(j, carry):
        base = sid * (NPAD // NS) + j * CH
        for g in range(CH // 16):
            idx_z[pl.ds(g * 16, 16)] = iota16 + (base + g * 16)
        pltpu.sync_copy(msg_rows, agg_acc.at[idx_z])
        return carry

    lax.fori_loop(0, (NPAD // NS) // CH, _zero_acc, 0)

    def _zero_denacc(j, carry):
        base = sid * (NPG // NS) + j * CH
        for g in range(CH // 16):
            idx_z[pl.ds(g * 16, 16)] = iota16 + (base + g * 16)
        pltpu.sync_copy(msg_rows, den_acc.at[idx_z])
        return carry

    lax.fori_loop(0, (NPG // NS) // CH, _zero_denacc, 0)

    pltpu.async_copy(amax_hbm, amax_vm, sem_a).wait()
    amax_v = amax_vm[...]
    maskb = iota16 < H

    plsc.subcore_barrier()

    assert NCHUNK % NW == 0
    nch = NCHUNK // NW

    def _chunk(i, carry):
        off = pl.multiple_of((wid + NW * i) * CH, 8)
        pltpu.async_copy(src_hbm.at[pl.ds(off, CH)], idx_s, sem_a).wait()
        pltpu.async_copy(dst_hbm.at[pl.ds(off, CH)], idx_d, sem_b).wait()
        ca = pltpu.async_copy(ts_hbm.at[idx_s], src_rows, sem_a)
        cb = pltpu.async_copy(td_hbm.at[idx_d], ald_rows, sem_b)
        dvv = idx_d[...]
        idx_dz[...] = lax.shift_right_logical(dvv, 3)
        ca.wait()
        cb.wait()

        for gi, width in ((0, 16),):
            dv = dvv & 7
            for j in range(width):
                k = gi + j
                als_v = src_rows[k, pl.ds(0, 16)]
                ald_v = ald_rows[k, pl.ds(0, 16)]
                t = als_v + ald_v
                e = jnp.maximum(t, 0.2 * t)
                u = amax_v + ald_v
                b = jnp.maximum(u, 0.2 * u)
                w = jnp.where(maskb, jnp.exp(e - b), 0.0)
                off16 = dv[j] * 16
                for h in range(H):
                    h_vec = jnp.full((16, 1), h, jnp.int32)
                    wspl = lax.gather(
                        w, h_vec,
                        lax.GatherDimensionNumbers(
                            offset_dims=(), collapsed_slice_dims=(0,),
                            start_index_map=(0,)),
                        slice_sizes=(1,),
                        mode=lax.GatherScatterMode.PROMISE_IN_BOUNDS)
                    part = src_rows[k, pl.ds(D + h * 16, 16)]
                    msg_rows[k, pl.ds(h * 16, 16)] = part * wspl
                for c in range(8):
                    den_rows[k, pl.ds(c * 16, 16)] = zero16
                den_rows[k, pl.ds(off16, 16)] = w

        pltpu.sync_copy(den_rows, den_acc.at[idx_dz], add=True)
        pltpu.sync_copy(msg_rows, agg_acc.at[idx_d], add=True)
        return carry

    lax.fori_loop(0, nch, _chunk, 0)

    plsc.subcore_barrier()

    def _wb(j, carry):
        base = sid * (NPAD // NS) + j * CH
        for g in range(CH // 16):
            idx_z[pl.ds(g * 16, 16)] = iota16 + (base + g * 16)
        pltpu.sync_copy(agg_acc.at[idx_z], msg_rows)
        hb = pl.multiple_of(base, 8)
        pltpu.sync_copy(msg_rows, agg_out.at[cid, pl.ds(hb, CH)])
        return carry

    lax.fori_loop(0, (NPAD // NS) // CH, _wb, 0)

    def _wb_den(j, carry):
        base = sid * (NPG // NS) + j * CH
        for g in range(CH // 16):
            idx_z[pl.ds(g * 16, 16)] = iota16 + (base + g * 16)
        pltpu.sync_copy(den_acc.at[idx_z], msg_rows)
        hb = pl.multiple_of(base, 8)
        pltpu.sync_copy(msg_rows, den_out.at[cid, pl.ds(hb, CH)])
        return carry

    lax.fori_loop(0, (NPG // NS) // CH, _wb_den, 0)


_edge_kernel = pl.kernel(
    _edge_body,
    out_type=(
        jax.ShapeDtypeStruct((NC, NPAD, D), f32),
        jax.ShapeDtypeStruct((NC, NPG, D), f32),
    ),
    mesh=plsc.VectorSubcoreMesh(core_axis_name="c", subcore_axis_name="s",
                                num_cores=NC, num_subcores=NS),
    scratch_types=(
        pltpu.VMEM((CH,), jnp.int32),
        pltpu.VMEM((CH,), jnp.int32),
        pltpu.VMEM((CH,), jnp.int32),
        pltpu.VMEM((CH,), jnp.int32),
        pltpu.VMEM((CH, TS), f32),
        pltpu.VMEM((CH, D), f32),
        pltpu.VMEM((CH, D), f32),
        pltpu.VMEM((CH, D), f32),
        pltpu.VMEM((16,), f32),
        pltpu.VMEM_SHARED((NPAD, D), f32),
        pltpu.VMEM_SHARED((NPG, D), f32),
        pltpu.SemaphoreType.DMA,
        pltpu.SemaphoreType.DMA,
    ),
)


def _attn_mat(a):
    m = (jnp.eye(H, dtype=f32)[:, None, :] * a[:, :, None]).reshape(D, H)
    return jnp.pad(m, ((0, 0), (0, DH - H)))


def _expand_mat():
    m = jnp.repeat(jnp.eye(H, dtype=f32), DH, axis=1)
    return jnp.pad(m, ((0, DH - H), (0, 0)))


def _edge_jnp(src, dst, ts, td, amax):
    als = ts[src, :DH]
    ald = td[dst, :DH]
    t = als + ald
    e = jnp.maximum(t, 0.2 * t)
    u = amax[None, :] + ald
    b = jnp.maximum(u, 0.2 * u)
    w = jnp.where(jnp.arange(DH) < H, jnp.exp(e - b), 0.0)
    wh = ts[src, D:]
    wexp = jnp.repeat(w[:, :H], DH, axis=1)
    agg = jax.ops.segment_sum(wh * wexp, dst, num_segments=N)
    den = jax.ops.segment_sum(w, dst, num_segments=N)
    aggp = jnp.stack([agg, jnp.zeros_like(agg)])
    denp = jnp.stack([den, jnp.zeros_like(den)])
    return aggp, denp


@jax.jit
def kernel(x, edge_index, W_in, b_in, W1, a_src1, a_dst1, W2, a_src2, a_dst2,
           W_out, b_out):
    src = edge_index[0]
    dst = edge_index[1]
    Exp = _expand_mat()

    ts1, td1, amax1 = _dense0(x, W_in, b_in.reshape(1, D), W1,
                              _attn_mat(a_src1), _attn_mat(a_dst1))
    agg1, den1p = _edge_kernel(src, dst, ts1, td1, amax1.reshape(DH))
    den1 = den1p.reshape(NC, NPAD, DH)[:, :N]
    ts2, td2, amax2 = _dense1(agg1[:, :N], den1, Exp, W2,
                              _attn_mat(a_src2), _attn_mat(a_dst2))
    agg2, den2p = _edge_kernel(src, dst, ts2, td2, amax2.reshape(DH))
    den2 = den2p.reshape(NC, NPAD, DH)[:, :N]
    return _dense2(agg2[:, :N], den2, Exp, W_out, b_out.reshape(1, OUT))


# speedup vs baseline: 195.7563x; 1.9502x over previous
import jax
import jax.numpy as jnp
from jax import lax
from jax.experimental import pallas as pl
from jax.experimental.pallas import tpu as pltpu
from jax.experimental.pallas import tpu_sc as plsc

N = 10000
E = 320000
D = 128
H = 8
DH = 16
OUT = 64
TS = 2 * D

NC = 2
NS = 16
NW = NC * NS
CH = 16
NCHUNK = E // CH
NPAD = 10240
NPG = NPAD // 8

_BLK = 1000
_GRID = N // _BLK

f32 = jnp.float32


def _emit_tables(wh, as_ref, ad_ref, ts_ref, td_ref, amax_ref):
    als = jnp.dot(wh, as_ref[...], preferred_element_type=f32)
    ald = jnp.dot(wh, ad_ref[...], preferred_element_type=f32)
    pad = jnp.zeros((_BLK, D - DH), f32)
    ts_ref[...] = jnp.concatenate([als, pad, wh], axis=1)
    td_ref[...] = jnp.concatenate([ald, pad], axis=1)
    blkmax = jnp.max(als, axis=0, keepdims=True)

    @pl.when(pl.program_id(0) == 0)
    def _():
        amax_ref[...] = blkmax

    @pl.when(pl.program_id(0) != 0)
    def _():
        amax_ref[...] = jnp.maximum(amax_ref[...], blkmax)


def _dense0_body(x_ref, win_ref, bin_ref, w1_ref, as_ref, ad_ref,
                 ts_ref, td_ref, amax_ref):
    h0 = jnp.dot(x_ref[...], win_ref[...], preferred_element_type=f32)
    h0 = h0 + bin_ref[...]
    wh = jnp.dot(h0, w1_ref[...], preferred_element_type=f32)
    _emit_tables(wh, as_ref, ad_ref, ts_ref, td_ref, amax_ref)


_TABLE_OUT_SPECS = [
    pl.BlockSpec((_BLK, TS), lambda i: (i, 0)),
    pl.BlockSpec((_BLK, D), lambda i: (i, 0)),
    pl.BlockSpec((1, DH), lambda i: (0, 0)),
]
_TABLE_OUT_SHAPE = [
    jax.ShapeDtypeStruct((N, TS), f32),
    jax.ShapeDtypeStruct((N, D), f32),
    jax.ShapeDtypeStruct((1, DH), f32),
]


def _dense0(x, W_in, b_in, W1, As, Ad):
    return pl.pallas_call(
        _dense0_body,
        grid=(_GRID,),
        in_specs=[
            pl.BlockSpec((_BLK, D), lambda i: (i, 0)),
            pl.BlockSpec((D, D), lambda i: (0, 0)),
            pl.BlockSpec((1, D), lambda i: (0, 0)),
            pl.BlockSpec((D, D), lambda i: (0, 0)),
            pl.BlockSpec((D, DH), lambda i: (0, 0)),
            pl.BlockSpec((D, DH), lambda i: (0, 0)),
        ],
        out_specs=_TABLE_OUT_SPECS,
        out_shape=_TABLE_OUT_SHAPE,
    )(x, W_in, b_in, W1, As, Ad)


def _dense1_body(agg_ref, den_ref, exp_ref, w2_ref, as_ref, ad_ref,
                 ts_ref, td_ref, amax_ref):
    agg = agg_ref[0] + agg_ref[1]
    den = den_ref[0] + den_ref[1]
    dexp = jnp.dot(den, exp_ref[...], preferred_element_type=f32)
    h = jnp.maximum(agg / (dexp + 1e-9), 0.0)
    wh = jnp.dot(h, w2_ref[...], preferred_element_type=f32)
    _emit_tables(wh, as_ref, ad_ref, ts_ref, td_ref, amax_ref)


def _dense1(agg, den, Exp, W2, As, Ad):
    return pl.pallas_call(
        _dense1_body,
        grid=(_GRID,),
        in_specs=[
            pl.BlockSpec((2, _BLK, D), lambda i: (0, i, 0)),
            pl.BlockSpec((2, _BLK, DH), lambda i: (0, i, 0)),
            pl.BlockSpec((DH, D), lambda i: (0, 0)),
            pl.BlockSpec((D, D), lambda i: (0, 0)),
            pl.BlockSpec((D, DH), lambda i: (0, 0)),
            pl.BlockSpec((D, DH), lambda i: (0, 0)),
        ],
        out_specs=_TABLE_OUT_SPECS,
        out_shape=_TABLE_OUT_SHAPE,
    )(agg, den, Exp, W2, As, Ad)


def _dense2_body(agg_ref, den_ref, exp_ref, wout_ref, bout_ref, o_ref):
    agg = agg_ref[0] + agg_ref[1]
    den = den_ref[0] + den_ref[1]
    dexp = jnp.dot(den, exp_ref[...], preferred_element_type=f32)
    h = jnp.maximum(agg / (dexp + 1e-9), 0.0)
    o_ref[...] = jnp.dot(h, wout_ref[...], preferred_element_type=f32) + bout_ref[...]


def _dense2(agg, den, Exp, W_out, b_out):
    return pl.pallas_call(
        _dense2_body,
        grid=(_GRID,),
        in_specs=[
            pl.BlockSpec((2, _BLK, D), lambda i: (0, i, 0)),
            pl.BlockSpec((2, _BLK, DH), lambda i: (0, i, 0)),
            pl.BlockSpec((DH, D), lambda i: (0, 0)),
            pl.BlockSpec((D, OUT), lambda i: (0, 0)),
            pl.BlockSpec((1, OUT), lambda i: (0, 0)),
        ],
        out_specs=pl.BlockSpec((_BLK, OUT), lambda i: (i, 0)),
        out_shape=jax.ShapeDtypeStruct((N, OUT), f32),
    )(agg, den, Exp, W_out, b_out)


def _edge_body(src_hbm, dst_hbm, ts_hbm, td_hbm, amax_hbm,
               agg_out, den_out,
               idx_s0, idx_s1, idx_d0, idx_d1, idx_dz, idx_z,
               src_rows0, src_rows1, ald_rows0, ald_rows1,
               msg_rows, den_rows, amax_vm, agg_acc, den_acc,
               isem0, isem1, gsem0, gsem1):
    cid = lax.axis_index("c")
    sid = lax.axis_index("s")
    wid = sid * NC + cid
    idx_s = (idx_s0, idx_s1)
    idx_d = (idx_d0, idx_d1)
    src_rows = (src_rows0, src_rows1)
    ald_rows = (ald_rows0, ald_rows1)
    isem = (isem0, isem1)
    gsem = (gsem0, gsem1)

    zero16 = jnp.zeros((16,), f32)
    iota16 = lax.broadcasted_iota(jnp.int32, (16,), 0)

    def _zero_msg(j, carry):
        for c in range(D // 16):
            msg_rows[j, pl.ds(c * 16, 16)] = zero16
        return carry

    lax.fori_loop(0, CH, _zero_msg, 0)

    def _zero_den(j, carry):
        for c in range(D // 16):
            den_rows[j, pl.ds(c * 16, 16)] = zero16
        return carry

    lax.fori_loop(0, CH, _zero_den, 0)

    def _zero_acc(j, carry):
        base = sid * (NPAD // NS) + j * CH
        for g in range(CH // 16):
            idx_z[pl.ds(g * 16, 16)] = iota16 + (base + g * 16)
        pltpu.sync_copy(msg_rows, agg_acc.at[idx_z])
        return carry

    lax.fori_loop(0, (NPAD // NS) // CH, _zero_acc, 0)

    def _zero_denacc(j, carry):
        base = sid * (NPG // NS) + j * CH
        for g in range(CH // 16):
            idx_z[pl.ds(g * 16, 16)] = iota16 + (base + g * 16)
        pltpu.sync_copy(msg_rows, den_acc.at[idx_z])
        return carry

    lax.fori_loop(0, (NPG // NS) // CH, _zero_denacc, 0)

    pltpu.async_copy(amax_hbm, amax_vm, isem0).wait()
    amax_v = amax_vm[...]
    maskb = iota16 < H

    plsc.subcore_barrier()

    assert NCHUNK % NW == 0
    nch = NCHUNK // NW

    def _off(i):
        return pl.multiple_of(
            jnp.minimum((wid + NW * i) * CH, E - CH), 8)

    def _issue_idx(i, p):
        off = _off(i)
        pltpu.async_copy(src_hbm.at[pl.ds(off, CH)], idx_s[p], isem[p])
        pltpu.async_copy(dst_hbm.at[pl.ds(off, CH)], idx_d[p], isem[p])

    def _drain_idx(p):
        pltpu.make_async_copy(src_hbm.at[pl.ds(0, CH)], idx_s[p], isem[p]).wait()
        pltpu.make_async_copy(dst_hbm.at[pl.ds(0, CH)], idx_d[p], isem[p]).wait()

    def _issue_gather(p):
        pltpu.async_copy(ts_hbm.at[idx_s[p]], src_rows[p], gsem[p])
        pltpu.async_copy(td_hbm.at[idx_d[p]], ald_rows[p], gsem[p])

    def _drain_gather(p):
        pltpu.make_async_copy(ts_hbm.at[idx_s[p]], src_rows[p], gsem[p]).wait()
        pltpu.make_async_copy(td_hbm.at[idx_d[p]], ald_rows[p], gsem[p]).wait()

    def _compute_scatter(p):
        dvv = idx_d[p][...]
        idx_dz[...] = lax.shift_right_logical(dvv, 3)
        dv = dvv & 7
        sr = src_rows[p]
        ar = ald_rows[p]
        for k in range(CH):
            als_v = sr[k, pl.ds(0, 16)]
            ald_v = ar[k, pl.ds(0, 16)]
            t = als_v + ald_v
            e = jnp.maximum(t, 0.2 * t)
            u = amax_v + ald_v
            b = jnp.maximum(u, 0.2 * u)
            w = jnp.where(maskb, jnp.exp(e - b), 0.0)
            off16 = dv[k] * 16
            for h in range(H):
                h_vec = jnp.full((16, 1), h, jnp.int32)
                wspl = lax.gather(
                    w, h_vec,
                    lax.GatherDimensionNumbers(
                        offset_dims=(), collapsed_slice_dims=(0,),
                        start_index_map=(0,)),
                    slice_sizes=(1,),
                    mode=lax.GatherScatterMode.PROMISE_IN_BOUNDS)
                part = sr[k, pl.ds(D + h * 16, 16)]
                msg_rows[k, pl.ds(h * 16, 16)] = part * wspl
            for c in range(8):
                den_rows[k, pl.ds(c * 16, 16)] = zero16
            den_rows[k, pl.ds(off16, 16)] = w
        pltpu.sync_copy(den_rows, den_acc.at[idx_dz], add=True)
        pltpu.sync_copy(msg_rows, agg_acc.at[idx_d[p]], add=True)

    def _body(i, p):
        _drain_gather(p)
        _drain_idx(1 - p)
        _issue_gather(1 - p)
        _compute_scatter(p)
        _issue_idx(i + 2, p)

    _issue_idx(0, 0)
    _drain_idx(0)
    _issue_gather(0)
    _issue_idx(1, 1)

    def _pair(j, carry):
        _body(2 * j, 0)
        _body(2 * j + 1, 1)
        return carry

    lax.fori_loop(0, (nch - 1) // 2, _pair, 0)

    _drain_gather(0)
    _compute_scatter(0)
    _drain_idx(1)

    plsc.subcore_barrier()

    def _wb(j, carry):
        base = sid * (NPAD // NS) + j * CH
        for g in range(CH // 16):
            idx_z[pl.ds(g * 16, 16)] = iota16 + (base + g * 16)
        pltpu.sync_copy(agg_acc.at[idx_z], msg_rows)
        hb = pl.multiple_of(base, 8)
        pltpu.sync_copy(msg_rows, agg_out.at[cid, pl.ds(hb, CH)])
        return carry

    lax.fori_loop(0, (NPAD // NS) // CH, _wb, 0)

    def _wb_den(j, carry):
        base = sid * (NPG // NS) + j * CH
        for g in range(CH // 16):
            idx_z[pl.ds(g * 16, 16)] = iota16 + (base + g * 16)
        pltpu.sync_copy(den_acc.at[idx_z], msg_rows)
        hb = pl.multiple_of(base, 8)
        pltpu.sync_copy(msg_rows, den_out.at[cid, pl.ds(hb, CH)])
        return carry

    lax.fori_loop(0, (NPG // NS) // CH, _wb_den, 0)


_edge_kernel = pl.kernel(
    _edge_body,
    out_type=(
        jax.ShapeDtypeStruct((NC, NPAD, D), f32),
        jax.ShapeDtypeStruct((NC, NPG, D), f32),
    ),
    mesh=plsc.VectorSubcoreMesh(core_axis_name="c", subcore_axis_name="s",
                                num_cores=NC, num_subcores=NS),
    scratch_types=(
        pltpu.VMEM((CH,), jnp.int32),
        pltpu.VMEM((CH,), jnp.int32),
        pltpu.VMEM((CH,), jnp.int32),
        pltpu.VMEM((CH,), jnp.int32),
        pltpu.VMEM((CH,), jnp.int32),
        pltpu.VMEM((CH,), jnp.int32),
        pltpu.VMEM((CH, TS), f32),
        pltpu.VMEM((CH, TS), f32),
        pltpu.VMEM((CH, D), f32),
        pltpu.VMEM((CH, D), f32),
        pltpu.VMEM((CH, D), f32),
        pltpu.VMEM((CH, D), f32),
        pltpu.VMEM((16,), f32),
        pltpu.VMEM_SHARED((NPAD, D), f32),
        pltpu.VMEM_SHARED((NPG, D), f32),
        pltpu.SemaphoreType.DMA,
        pltpu.SemaphoreType.DMA,
        pltpu.SemaphoreType.DMA,
        pltpu.SemaphoreType.DMA,
    ),
)


def _attn_mat(a):
    m = (jnp.eye(H, dtype=f32)[:, None, :] * a[:, :, None]).reshape(D, H)
    return jnp.pad(m, ((0, 0), (0, DH - H)))


def _expand_mat():
    m = jnp.repeat(jnp.eye(H, dtype=f32), DH, axis=1)
    return jnp.pad(m, ((0, DH - H), (0, 0)))


def _edge_jnp(src, dst, ts, td, amax):
    als = ts[src, :DH]
    ald = td[dst, :DH]
    t = als + ald
    e = jnp.maximum(t, 0.2 * t)
    u = amax[None, :] + ald
    b = jnp.maximum(u, 0.2 * u)
    w = jnp.where(jnp.arange(DH) < H, jnp.exp(e - b), 0.0)
    wh = ts[src, D:]
    wexp = jnp.repeat(w[:, :H], DH, axis=1)
    agg = jax.ops.segment_sum(wh * wexp, dst, num_segments=N)
    den = jax.ops.segment_sum(w, dst, num_segments=N)
    aggp = jnp.stack([agg, jnp.zeros_like(agg)])
    denp = jnp.stack([den, jnp.zeros_like(den)])
    return aggp, denp


@jax.jit
def kernel(x, edge_index, W_in, b_in, W1, a_src1, a_dst1, W2, a_src2, a_dst2,
           W_out, b_out):
    src = edge_index[0]
    dst = edge_index[1]
    Exp = _expand_mat()

    ts1, td1, amax1 = _dense0(x, W_in, b_in.reshape(1, D), W1,
                              _attn_mat(a_src1), _attn_mat(a_dst1))
    agg1, den1p = _edge_kernel(src, dst, ts1, td1, amax1.reshape(DH))
    den1 = den1p.reshape(NC, NPAD, DH)[:, :N]
    ts2, td2, amax2 = _dense1(agg1[:, :N], den1, Exp, W2,
                              _attn_mat(a_src2), _attn_mat(a_dst2))
    agg2, den2p = _edge_kernel(src, dst, ts2, td2, amax2.reshape(DH))
    den2 = den2p.reshape(NC, NPAD, DH)[:, :N]
    return _dense2(agg2[:, :N], den2, Exp, W_out, b_out.reshape(1, OUT))
